# Initial kernel scaffold; baseline (speedup 1.0000x reference)
import jax
import jax.numpy as jnp
from jax.experimental import pallas as pl


def kernel(x, table, W_ih_f, W_hh_f, b_ih_f, b_hh_f, W_ih_b, W_hh_b, b_ih_b, b_hh_b):
    raise NotImplementedError("write your pallas kernel here")


# speedup vs baseline: 2.5216x; 2.5216x over previous
import functools

import jax
import jax.numpy as jnp
from jax import lax
from jax.experimental import pallas as pl
from jax.experimental.pallas import tpu as pltpu
from jax.experimental.pallas import tpu_sc as plsc

_NUM_SC = 2
_NUM_TEC = 16
_NUM_WORKERS = _NUM_SC * _NUM_TEC

_GATHER_CHUNK = 128


def _sc_gather(table2, idx_flat):
    n_rows = idx_flat.shape[0]
    width = table2.shape[1]
    per_w = n_rows // _NUM_WORKERS
    n_chunks = per_w // _GATHER_CHUNK

    mesh = plsc.VectorSubcoreMesh(core_axis_name="c", subcore_axis_name="s")

    @functools.partial(
        pl.kernel,
        mesh=mesh,
        out_type=jax.ShapeDtypeStruct((n_rows, width), jnp.float32),
        scratch_types=[
            pltpu.VMEM((_GATHER_CHUNK,), jnp.int32),
            pltpu.VMEM((_GATHER_CHUNK, width), jnp.float32),
            pltpu.SemaphoreType.DMA,
        ],
    )
    def gather_kernel(table_hbm, idx_hbm, out_hbm, idx_v, rows_v, sem):
        wid = lax.axis_index("s") * _NUM_SC + lax.axis_index("c")
        base = wid * per_w

        def body(i, carry):
            off = base + i * _GATHER_CHUNK
            pltpu.sync_copy(idx_hbm.at[pl.ds(off, _GATHER_CHUNK)], idx_v)
            pltpu.async_copy(table_hbm.at[idx_v], rows_v, sem).wait()
            pltpu.sync_copy(rows_v, out_hbm.at[pl.ds(off, _GATHER_CHUNK)])
            return carry

        lax.fori_loop(0, n_chunks, body, 0)

    return gather_kernel(table2, idx_flat)


_TT = 2


def _bigru(emb2, parB, wif, whf, bif, bhf, wib, whb, bib, bhb,
           interpret=False):
    t_len, b, two_e = emb2.shape
    e = two_e // 2
    h = whf.shape[0]
    nt = t_len // _TT

    def body(xf_ref, xb_ref, pf_ref, pb_ref,
             wif_ref, whf_ref, bif_ref, bhf_ref,
             wib_ref, whb_ref, bib_ref, bhb_ref,
             outf_ref, outb_ref, hf_ref, hb_ref):
        @pl.when(pl.program_id(0) == 0)
        def _():
            hf_ref[...] = jnp.zeros_like(hf_ref)
            hb_ref[...] = jnp.zeros_like(hb_ref)

        w_if = wif_ref[...]
        w_hf = whf_ref[...]
        b_if = bif_ref[...]
        b_hf = bhf_ref[...]
        w_ib = wib_ref[...]
        w_hb = whb_ref[...]
        b_ib = bib_ref[...]
        b_hb = bhb_ref[...]

        def pick(pair, p_col):
            return jnp.where(p_col > 0.5, pair[:, e:], pair[:, :e])

        def gru_step(x_t, h_prev, w_i, w_h, b_i, b_h):
            gi = jnp.dot(x_t, w_i, preferred_element_type=jnp.float32) + b_i
            gh = jnp.dot(h_prev, w_h, preferred_element_type=jnp.float32) + b_h
            r = jax.nn.sigmoid(gi[:, :h] + gh[:, :h])
            z = jax.nn.sigmoid(gi[:, h:2 * h] + gh[:, h:2 * h])
            n = jnp.tanh(gi[:, 2 * h:] + r * gh[:, 2 * h:])
            return n + z * (h_prev - n)

        for i in range(_TT):
            x_f = pick(xf_ref[i], pf_ref[0, :, i:i + 1])
            h_f = gru_step(x_f, hf_ref[...], w_if, w_hf, b_if, b_hf)
            hf_ref[...] = h_f
            outf_ref[i] = h_f

            x_b = pick(xb_ref[_TT - 1 - i], pb_ref[0, :, _TT - 1 - i:_TT - i])
            h_b = gru_step(x_b, hb_ref[...], w_ib, w_hb, b_ib, b_hb)
            hb_ref[...] = h_b
            outb_ref[_TT - 1 - i] = h_b

    full = lambda shape: pl.BlockSpec(shape, lambda j: (0,) * len(shape))
    out_f, out_b = pl.pallas_call(
        body,
        grid=(nt,),
        in_specs=[
            pl.BlockSpec((_TT, b, two_e), lambda j: (j, 0, 0)),
            pl.BlockSpec((_TT, b, two_e), lambda j: (nt - 1 - j, 0, 0)),
            pl.BlockSpec((1, b, _TT), lambda j: (j, 0, 0)),
            pl.BlockSpec((1, b, _TT), lambda j: (nt - 1 - j, 0, 0)),
            full((e, 3 * h)), full((h, 3 * h)), full((1, 3 * h)), full((1, 3 * h)),
            full((e, 3 * h)), full((h, 3 * h)), full((1, 3 * h)), full((1, 3 * h)),
        ],
        out_specs=[
            pl.BlockSpec((_TT, b, h), lambda j: (j, 0, 0)),
            pl.BlockSpec((_TT, b, h), lambda j: (nt - 1 - j, 0, 0)),
        ],
        out_shape=[
            jax.ShapeDtypeStruct((t_len, b, h), jnp.float32),
            jax.ShapeDtypeStruct((t_len, b, h), jnp.float32),
        ],
        scratch_shapes=[
            pltpu.VMEM((b, h), jnp.float32),
            pltpu.VMEM((b, h), jnp.float32),
        ],
        interpret=interpret,
    )(emb2, emb2, parB, parB, wif, whf, bif, bhf, wib, whb, bib, bhb)
    return out_f, out_b


def kernel(x, table, W_ih_f, W_hh_f, b_ih_f, b_hh_f,
           W_ih_b, W_hh_b, b_ih_b, b_hh_b):
    b, t_len = x.shape
    e = table.shape[1]
    h = W_hh_f.shape[1]

    x = x.astype(jnp.int32)
    table2 = table.reshape(-1, 2 * e)
    idx2 = jnp.transpose(x).reshape(-1) >> 1
    nt = t_len // _TT
    parB = (x & 1).astype(jnp.float32).reshape(b, nt, _TT).transpose(1, 0, 2)

    emb2 = _sc_gather(table2, idx2).reshape(t_len, b, 2 * e)

    out_f, out_b = _bigru(
        emb2, parB,
        jnp.transpose(W_ih_f), jnp.transpose(W_hh_f),
        b_ih_f.reshape(1, 3 * h), b_hh_f.reshape(1, 3 * h),
        jnp.transpose(W_ih_b), jnp.transpose(W_hh_b),
        b_ih_b.reshape(1, 3 * h), b_hh_b.reshape(1, 3 * h),
    )
    out = jnp.concatenate([out_f, out_b], axis=-1)
    return jnp.transpose(out, (1, 0, 2))


# speedup vs baseline: 2.9153x; 1.1561x over previous
import functools

import jax
import jax.numpy as jnp
from jax import lax
from jax.experimental import pallas as pl
from jax.experimental.pallas import tpu as pltpu
from jax.experimental.pallas import tpu_sc as plsc

_NUM_SC = 2
_NUM_TEC = 16
_NUM_WORKERS = _NUM_SC * _NUM_TEC

_GATHER_CHUNK = 128


def _sc_gather(table, idx_flat):
    n_rows = idx_flat.shape[0]
    width = table.shape[1]
    per_w = n_rows // _NUM_WORKERS
    n_chunks = per_w // _GATHER_CHUNK

    mesh = plsc.VectorSubcoreMesh(core_axis_name="c", subcore_axis_name="s")

    @functools.partial(
        pl.kernel,
        mesh=mesh,
        out_type=jax.ShapeDtypeStruct((n_rows, width), jnp.float32),
        scratch_types=[
            pltpu.VMEM((_GATHER_CHUNK,), jnp.int32),
            pltpu.VMEM((_GATHER_CHUNK, width), jnp.float32),
            pltpu.SemaphoreType.DMA,
        ],
        compiler_params=pltpu.CompilerParams(use_tc_tiling_on_sc=False),
    )
    def gather_kernel(table_hbm, idx_hbm, out_hbm, idx_v, rows_v, sem):
        wid = lax.axis_index("s") * _NUM_SC + lax.axis_index("c")
        base = wid * per_w

        def body(i, carry):
            off = base + i * _GATHER_CHUNK
            pltpu.sync_copy(idx_hbm.at[pl.ds(off, _GATHER_CHUNK)], idx_v)
            pltpu.async_copy(table_hbm.at[idx_v], rows_v, sem).wait()
            pltpu.sync_copy(rows_v, out_hbm.at[pl.ds(off, _GATHER_CHUNK)])
            return carry

        lax.fori_loop(0, n_chunks, body, 0)

    return gather_kernel(table, idx_flat)


_TT = 2


def _bigru(embT, wrz_f, wn_f, urz_f, un_f, brz_f, bn_f, bhn_f,
           wrz_b, wn_b, urz_b, un_b, brz_b, bn_b, bhn_b, interpret=False):
    t_len, b, e = embT.shape
    h = un_f.shape[0]
    nt = t_len // _TT

    def body(xf_ref, xb_ref,
             wrzf_ref, wnf_ref, urzf_ref, unf_ref, brzf_ref, bnf_ref, bhnf_ref,
             wrzb_ref, wnb_ref, urzb_ref, unb_ref, brzb_ref, bnb_ref, bhnb_ref,
             outf_ref, outb_ref, hf_ref, hb_ref):
        @pl.when(pl.program_id(0) == 0)
        def _():
            hf_ref[...] = jnp.zeros_like(hf_ref)
            hb_ref[...] = jnp.zeros_like(hb_ref)

        def dot(a, w):
            return jnp.dot(a, w, preferred_element_type=jnp.float32)

        def gru_step(x_t, h_prev, wrz, wn, urz, un, brz, bn, bhn):
            rz = jax.nn.sigmoid(dot(x_t, wrz) + dot(h_prev, urz) + brz)
            r = rz[:, :h]
            z = rz[:, h:]
            hn = dot(h_prev, un) + bhn
            n = jnp.tanh(dot(x_t, wn) + bn + r * hn)
            return n + z * (h_prev - n)

        for i in range(_TT):
            h_f = gru_step(xf_ref[i], hf_ref[...], wrzf_ref[...], wnf_ref[...],
                           urzf_ref[...], unf_ref[...], brzf_ref[...],
                           bnf_ref[...], bhnf_ref[...])
            hf_ref[...] = h_f
            outf_ref[i] = h_f

            h_b = gru_step(xb_ref[_TT - 1 - i], hb_ref[...], wrzb_ref[...],
                           wnb_ref[...], urzb_ref[...], unb_ref[...],
                           brzb_ref[...], bnb_ref[...], bhnb_ref[...])
            hb_ref[...] = h_b
            outb_ref[_TT - 1 - i] = h_b

    full = lambda shape: pl.BlockSpec(shape, lambda j: (0,) * len(shape))
    wspecs = [full((e, 2 * h)), full((e, h)), full((h, 2 * h)), full((h, h)),
              full((1, 2 * h)), full((1, h)), full((1, h))]
    out_f, out_b = pl.pallas_call(
        body,
        grid=(nt,),
        in_specs=[
            pl.BlockSpec((_TT, b, e), lambda j: (j, 0, 0)),
            pl.BlockSpec((_TT, b, e), lambda j: (nt - 1 - j, 0, 0)),
        ] + wspecs + wspecs,
        out_specs=[
            pl.BlockSpec((_TT, b, h), lambda j: (j, 0, 0)),
            pl.BlockSpec((_TT, b, h), lambda j: (nt - 1 - j, 0, 0)),
        ],
        out_shape=[
            jax.ShapeDtypeStruct((t_len, b, h), jnp.float32),
            jax.ShapeDtypeStruct((t_len, b, h), jnp.float32),
        ],
        scratch_shapes=[
            pltpu.VMEM((b, h), jnp.float32),
            pltpu.VMEM((b, h), jnp.float32),
        ],
        interpret=interpret,
    )(embT, embT,
      wrz_f, wn_f, urz_f, un_f, brz_f, bn_f, bhn_f,
      wrz_b, wn_b, urz_b, un_b, brz_b, bn_b, bhn_b)
    return out_f, out_b


def _prep_weights(W_ih, W_hh, b_ih, b_hh, h):
    wi = jnp.transpose(W_ih)
    wh = jnp.transpose(W_hh)
    wrz = wi[:, :2 * h]
    wn = wi[:, 2 * h:]
    urz = wh[:, :2 * h]
    un = wh[:, 2 * h:]
    brz = (b_ih[:2 * h] + b_hh[:2 * h]).reshape(1, 2 * h)
    bn = b_ih[2 * h:].reshape(1, h)
    bhn = b_hh[2 * h:].reshape(1, h)
    return wrz, wn, urz, un, brz, bn, bhn


def kernel(x, table, W_ih_f, W_hh_f, b_ih_f, b_hh_f,
           W_ih_b, W_hh_b, b_ih_b, b_hh_b):
    b, t_len = x.shape
    e = table.shape[1]
    h = W_hh_f.shape[1]

    x = x.astype(jnp.int32)
    idx = jnp.transpose(x).reshape(-1)
    embT = _sc_gather(table, idx).reshape(t_len, b, e)

    out_f, out_b = _bigru(
        embT,
        *_prep_weights(W_ih_f, W_hh_f, b_ih_f, b_hh_f, h),
        *_prep_weights(W_ih_b, W_hh_b, b_ih_b, b_hh_b, h),
    )
    out = jnp.concatenate([out_f, out_b], axis=-1)
    return jnp.transpose(out, (1, 0, 2))
